# baseline (speedup 1.0000x reference)
import functools

import jax
import jax.numpy as jnp
from jax import lax
from jax.experimental import pallas as pl
from jax.experimental.pallas import tpu as pltpu
from jax.experimental.pallas import tpu_sc as plsc

_NC = 2
_NS = 16
_NW = _NC * _NS
_CHUNK = 640
_DP = 128


def _gather_kernel(n_chunks, b_per_w,
                   idx_hbm, table_hbm, out_hbm,
                   idx_all, rows0, rows1, sg0, sg1, ss0, ss1):
    wid = lax.axis_index("s") * _NC + lax.axis_index("c")
    base = wid * b_per_w
    rows = (rows0, rows1)
    sg = (sg0, sg1)
    ss = (ss0, ss1)

    pltpu.sync_copy(idx_hbm.at[pl.ds(base, b_per_w)], idx_all)

    def gather_copy(c, b):
        return pltpu.make_async_copy(
            table_hbm.at[idx_all.at[pl.ds(c * _CHUNK, _CHUNK)]], rows[b], sg[b])

    def store_copy(c, b):
        return pltpu.make_async_copy(
            rows[b],
            out_hbm.at[pl.ds(base + c * _CHUNK, _CHUNK), pl.ds(0, 64)],
            ss[b])

    gather_copy(0, 0).start()
    gather_copy(1, 1).start()

    def body(g, carry):
        c0 = 2 * g
        for b in (0, 1):
            gather_copy(c0 + b, b).wait()
            store_copy(c0 + b, b).start()
        for b in (0, 1):
            store_copy(c0 + b, b).wait()
            gather_copy(c0 + 2 + b, b).start()
        return carry

    n_groups = n_chunks // 2
    lax.fori_loop(0, n_groups - 1, body, 0)

    c0 = n_chunks - 2
    for b in (0, 1):
        gather_copy(c0 + b, b).wait()
        store_copy(c0 + b, b).start()
    for b in (0, 1):
        store_copy(c0 + b, b).wait()


def kernel(token_ids, weight):
    bsz, seq = token_ids.shape
    nv, d = weight.shape
    n = bsz * seq
    wpad = jnp.pad(weight, ((0, 0), (0, d))).reshape(2 * nv, d)
    idx_flat = (token_ids.reshape(n) * 2).astype(jnp.int32)

    b_per_w = n // _NW
    n_chunks = b_per_w // _CHUNK

    mesh = plsc.VectorSubcoreMesh(core_axis_name="c", subcore_axis_name="s")
    k = functools.partial(
        pl.kernel,
        mesh=mesh,
        out_type=jax.ShapeDtypeStruct((n, _DP), jnp.float32),
        scratch_types=[
            pltpu.VMEM((b_per_w,), jnp.int32),
            pltpu.VMEM((_CHUNK, d), jnp.float32),
            pltpu.VMEM((_CHUNK, d), jnp.float32),
            pltpu.SemaphoreType.DMA,
            pltpu.SemaphoreType.DMA,
            pltpu.SemaphoreType.DMA,
            pltpu.SemaphoreType.DMA,
        ],
        compiler_params=pltpu.CompilerParams(use_tc_tiling_on_sc=False),
    )(functools.partial(_gather_kernel, n_chunks, b_per_w))

    out = k(idx_flat, wpad)
    return out[:, :d].reshape(bsz, seq, d)


# speedup vs baseline: 1.0039x; 1.0039x over previous
import functools

import jax
import jax.numpy as jnp
from jax import lax
from jax.experimental import pallas as pl
from jax.experimental.pallas import tpu as pltpu
from jax.experimental.pallas import tpu_sc as plsc

_NC = 2
_NS = 16
_NW = _NC * _NS
_CHUNK = 512
_DP = 128


def _gather_kernel(n_chunks, b_per_w,
                   idx_hbm, table_hbm, out_hbm,
                   idx_all, rows0, rows1, sg0, sg1, ss0, ss1):
    wid = lax.axis_index("s") * _NC + lax.axis_index("c")
    base = wid * b_per_w
    rows = (rows0, rows1)
    sg = (sg0, sg1)
    ss = (ss0, ss1)

    pltpu.sync_copy(idx_hbm.at[pl.ds(base, b_per_w)], idx_all)

    def gather_copy(c, b):
        return pltpu.make_async_copy(
            table_hbm.at[idx_all.at[pl.ds(c * _CHUNK, _CHUNK)]], rows[b], sg[b])

    def store_copy(c, b):
        return pltpu.make_async_copy(
            rows[b],
            out_hbm.at[pl.ds(base + c * _CHUNK, _CHUNK), pl.ds(0, 64)],
            ss[b])

    gather_copy(0, 0).start()
    gather_copy(1, 1).start()

    def body(g, carry):
        c0 = 2 * g
        for b in (0, 1):
            gather_copy(c0 + b, b).wait()
            store_copy(c0 + b, b).start()
        for b in (0, 1):
            store_copy(c0 + b, b).wait()
            gather_copy(c0 + 2 + b, b).start()
        return carry

    n_groups = n_chunks // 2
    lax.fori_loop(0, n_groups - 1, body, 0)

    c0 = n_chunks - 2
    for b in (0, 1):
        gather_copy(c0 + b, b).wait()
        store_copy(c0 + b, b).start()
    for b in (0, 1):
        store_copy(c0 + b, b).wait()


def kernel(token_ids, weight):
    bsz, seq = token_ids.shape
    nv, d = weight.shape
    n = bsz * seq
    wpad = jnp.pad(weight, ((0, 0), (0, d))).reshape(2 * nv, d)
    idx_flat = (token_ids.reshape(n) * 2).astype(jnp.int32)

    b_per_w = n // _NW
    n_chunks = b_per_w // _CHUNK

    mesh = plsc.VectorSubcoreMesh(core_axis_name="c", subcore_axis_name="s")
    k = functools.partial(
        pl.kernel,
        mesh=mesh,
        out_type=jax.ShapeDtypeStruct((n, _DP), jnp.float32),
        scratch_types=[
            pltpu.VMEM((b_per_w,), jnp.int32),
            pltpu.VMEM((_CHUNK, d), jnp.float32),
            pltpu.VMEM((_CHUNK, d), jnp.float32),
            pltpu.SemaphoreType.DMA,
            pltpu.SemaphoreType.DMA,
            pltpu.SemaphoreType.DMA,
            pltpu.SemaphoreType.DMA,
        ],
        compiler_params=pltpu.CompilerParams(use_tc_tiling_on_sc=False),
    )(functools.partial(_gather_kernel, n_chunks, b_per_w))

    out = k(idx_flat, wpad)
    return out[:, :d].reshape(bsz, seq, d)
